# baseline (speedup 1.0000x reference)
import functools

import jax
import jax.numpy as jnp
from jax import lax
from jax.experimental import pallas as pl
from jax.experimental.pallas import tpu as pltpu
from jax.experimental.pallas import tpu_sc as plsc

EMB_DIM = 64
BATCH = 16384
L = 16
NC = 2
NS = 16
NW = NC * NS
B_PER_W = BATCH // NW
CHUNK = 128
N_CHUNKS = B_PER_W // CHUNK
BLOCKS = B_PER_W // L


@functools.partial(
    pl.kernel,
    out_type=jax.ShapeDtypeStruct((BATCH,), jnp.float32),
    mesh=plsc.VectorSubcoreMesh(core_axis_name="c", subcore_axis_name="s"),
    compiler_params=pltpu.CompilerParams(
        needs_layout_passes=False, use_tc_tiling_on_sc=False),
    scratch_types=[
        pltpu.VMEM((N_CHUNKS, CHUNK), jnp.int32),
        pltpu.VMEM((N_CHUNKS, CHUNK), jnp.int32),
        pltpu.VMEM((B_PER_W, EMB_DIM), jnp.float32),
        pltpu.VMEM((B_PER_W, EMB_DIM), jnp.float32),
        pltpu.VMEM((B_PER_W,), jnp.float32),
        pltpu.VMEM((L * L,), jnp.float32),
        pltpu.SemaphoreType.DMA,
    ],
)
def _two_tower_sc(u_hbm, p_hbm, ue_hbm, pe_hbm, out_hbm,
                  uidx_v, pidx_v, urows_v, prows_v, out_v, tr_v, sem):
    wid = lax.axis_index("s") * NC + lax.axis_index("c")
    base = wid * B_PER_W

    pltpu.sync_copy(u_hbm.at[wid], uidx_v)
    pltpu.sync_copy(p_hbm.at[wid], pidx_v)

    copies = []
    for j in range(N_CHUNKS):
        copies.append(pltpu.async_copy(
            ue_hbm.at[uidx_v.at[j]], urows_v.at[pl.ds(j * CHUNK, CHUNK)], sem))
        copies.append(pltpu.async_copy(
            pe_hbm.at[pidx_v.at[j]], prows_v.at[pl.ds(j * CHUNK, CHUNK)], sem))
    for c in copies:
        c.wait()

    lane_ids = lax.iota(jnp.int32, L)

    def body(b, carry):
        for i in range(L):
            rr = b * L + i
            urow = urows_v.at[rr]
            prow = prows_v.at[rr]
            t0 = urow[pl.ds(0, L)] * prow[pl.ds(0, L)]
            t1 = urow[pl.ds(L, L)] * prow[pl.ds(L, L)]
            t2 = urow[pl.ds(2 * L, L)] * prow[pl.ds(2 * L, L)]
            t3 = urow[pl.ds(3 * L, L)] * prow[pl.ds(3 * L, L)]
            part = (t0 + t1) + (t2 + t3)
            plsc.store_scatter(tr_v, [lane_ids * L + i], part)
        dot = tr_v[pl.ds(0, L)]
        for l in range(1, L):
            dot = dot + tr_v[pl.ds(l * L, L)]
        out_v[pl.ds(b * L, L)] = 1.0 / (1.0 + jnp.exp(-dot))
        return carry

    lax.fori_loop(0, BLOCKS, body, 0)

    pltpu.sync_copy(out_v, out_hbm.at[pl.ds(base, B_PER_W)])


def kernel(u, p, user_emb, prod_emb):
    u3 = u.astype(jnp.int32).reshape(NW, N_CHUNKS, CHUNK)
    p3 = p.astype(jnp.int32).reshape(NW, N_CHUNKS, CHUNK)
    return _two_tower_sc(u3, p3, user_emb, prod_emb)
